# Initial kernel scaffold; baseline (speedup 1.0000x reference)
import jax
import jax.numpy as jnp
from jax.experimental import pallas as pl


def kernel(x, linear_table, cross_tables, bias):
    raise NotImplementedError("write your pallas kernel here")


# speedup vs baseline: 22.3735x; 22.3735x over previous
import functools

import numpy as np
import jax
import jax.numpy as jnp
from jax import lax
from jax.experimental import pallas as pl
from jax.experimental.pallas import tpu as pltpu
from jax.experimental.pallas import tpu_sc as plsc

_NF = 26
_D = 16
_VOC = 1000
_TOT = _NF * _VOC
_B = 4096

_PI, _PJ = np.triu_indices(_NF, 1)
_NPAIR = len(_PI)
_NW = 32
_BPT = _B // _NW
_EG = 4
_NG = _BPT // _EG
_TB = 104
_GR = _NF * _TB

_SEL2 = np.zeros((_NF, 32), np.int32)
for _t in range(_NF):
    _SEL2[_t, :25] = [f for f in range(_NF) if f != _t]

_mesh = plsc.VectorSubcoreMesh(core_axis_name="c", subcore_axis_name="s")


@functools.partial(
    pl.kernel,
    mesh=_mesh,
    compiler_params=pltpu.CompilerParams(
        needs_layout_passes=False, use_tc_tiling_on_sc=False),
    out_type=jax.ShapeDtypeStruct((_B,), jnp.float32),
    scratch_types=[
        pltpu.VMEM((2, _GR, _D), jnp.float32),
        pltpu.VMEM((_GR,), jnp.int32),
        pltpu.VMEM((_GR,), jnp.int32),
        pltpu.VMEM((_BPT * 32,), jnp.int32),
        pltpu.VMEM((_NF * 32,), jnp.int32),
        pltpu.VMEM((2, _EG * 32), jnp.float32),
        pltpu.VMEM((_BPT,), jnp.float32),
        pltpu.SemaphoreType.DMA,
        pltpu.SemaphoreType.DMA,
    ],
)
def _ffm_sc(tab_hbm, xo_hbm, lin_hbm, sel_hbm, out_hbm,
            rows_v, idx0_v, idx1_v, xo_v, sel_v, linrows_v, out_v,
            gsem0, gsem1):
    wid = lax.axis_index("s") * 2 + lax.axis_index("c")
    gsems = (gsem0, gsem1)
    idxbufs = (idx0_v, idx1_v)

    pltpu.sync_copy(xo_hbm.at[pl.ds(wid * (_BPT * 32), _BPT * 32)], xo_v)
    pltpu.sync_copy(sel_hbm, sel_v)

    lanes = lax.iota(jnp.int32, 16)
    lin_mask = lanes < (_NF - 16)
    zi16 = jnp.zeros((16,), jnp.int32)

    def zinit(k, c):
        idx0_v[pl.ds(k * 16, 16)] = zi16
        idx1_v[pl.ds(k * 16, 16)] = zi16
        return c

    lax.fori_loop(0, _GR // 16, zinit, 0)

    def build_idx(g, b):
        gc = jnp.minimum(g, _NG - 1)
        ib = idxbufs[b]
        for t in range(_NF):
            s0 = sel_v[pl.ds(t * 32, 16)]
            s1 = sel_v[pl.ds(t * 32 + 16, 16)]
            for e in range(_EG):
                le32 = jnp.full((16,), 32, jnp.int32) * (gc * _EG + e)
                v0 = plsc.load_gather(xo_v, [s0 + le32])
                v1 = plsc.load_gather(xo_v, [s1 + le32])
                base = t * _TB + e * 25
                plsc.store_scatter(ib, [lanes + base], v0)
                plsc.store_scatter(ib, [lanes + (base + 16)], v1,
                                   mask=lanes < 9)

    def fire_gather(g, b):
        gc = jnp.minimum(g, _NG - 1)
        pltpu.async_copy(
            lin_hbm.at[xo_v.at[pl.ds(gc * (_EG * 32), _EG * 32)]],
            linrows_v.at[b], gsems[b])
        for t in range(_NF):
            pltpu.async_copy(
                tab_hbm.at[t].at[idxbufs[b].at[pl.ds(t * _TB, _TB)]],
                rows_v.at[b, pl.ds(t * _TB, _TB)], gsems[b])

    def wait_gather(b):
        pltpu.make_async_copy(
            lin_hbm.at[xo_v.at[pl.ds(0, _EG * 32)]],
            linrows_v.at[b], gsems[b]).wait()
        for t in range(_NF):
            pltpu.make_async_copy(
                tab_hbm.at[t].at[idxbufs[b].at[pl.ds(t * _TB, _TB)]],
                rows_v.at[b, pl.ds(t * _TB, _TB)], gsems[b]).wait()

    def compute(g, b):
        def elem(e, carry):
            e25 = e * 25
            accs = [jnp.zeros((_D,), jnp.float32) for _ in range(4)]
            for p in range(_NPAIR):
                i = int(_PI[p])
                j = int(_PJ[p])
                rij = rows_v[b, e25 + j * _TB + i]
                rji = rows_v[b, e25 + i * _TB + (j - 1)]
                accs[p % 4] = accs[p % 4] + rij * rji
            acc = (accs[0] + accs[1]) + (accs[2] + accs[3])
            l0 = linrows_v[b, pl.ds(e * 32, 16)]
            l1 = linrows_v[b, pl.ds(e * 32 + 16, 16)]
            zv = acc + l0 + jnp.where(lin_mask, l1, 0.0)
            zc = plsc.cumsum(zv)
            le = g * _EG + e
            plsc.store_scatter(out_v, [jnp.full((16,), le, jnp.int32)],
                               zc, mask=lanes == 15)
            return carry

        lax.fori_loop(0, _EG, elem, 0)

    build_idx(0, 0)
    fire_gather(0, 0)

    def pair_body(gg, carry):
        for bb in range(2):
            g = 2 * gg + bb
            nb = 1 - bb
            build_idx(g + 1, nb)
            fire_gather(g + 1, nb)
            wait_gather(bb)
            compute(g, bb)
        return carry

    lax.fori_loop(0, _NG // 2, pair_body, 0)
    wait_gather(0)

    for k in range(_BPT // 16):
        z = out_v[pl.ds(k * 16, 16)]
        out_v[pl.ds(k * 16, 16)] = 1.0 / (1.0 + jnp.exp(-z))
    pltpu.sync_copy(out_v, out_hbm.at[pl.ds(wid * _BPT, _BPT)])


def kernel(x, linear_table, cross_tables, bias):
    offsets = jnp.asarray(np.arange(_NF, dtype=np.int32) * _VOC)
    xo = x + offsets[None, :]
    xof = jnp.pad(xo, ((0, 0), (0, 32 - _NF))).reshape(-1)
    lin2 = linear_table[:, 0] + bias[0] / float(_NF)
    sel2 = jnp.asarray(_SEL2.reshape(-1))
    out = _ffm_sc(cross_tables, xof, lin2, sel2)
    return out[:, None]


# speedup vs baseline: 26.7322x; 1.1948x over previous
import functools

import numpy as np
import jax
import jax.numpy as jnp
from jax import lax
from jax.experimental import pallas as pl
from jax.experimental.pallas import tpu as pltpu
from jax.experimental.pallas import tpu_sc as plsc

_NF = 26
_D = 16
_VOC = 1000
_TOT = _NF * _VOC
_B = 4096

_PI, _PJ = np.triu_indices(_NF, 1)
_NPAIR = len(_PI)
_ROWS = 2 * _NPAIR
_RPE = 656
_NW = 32
_BPT = _B // _NW
_EG = 4
_NG = _BPT // _EG
_GW = _EG * _RPE
_CHUNKS = [(0, 128), (128, 128), (256, 128), (384, 128), (512, 128), (640, 16)]

_SEL = np.zeros(_RPE, np.int32)
_ADD = np.zeros(_RPE, np.int32)
_SEL[0:_ROWS:2] = _PI
_ADD[0:_ROWS:2] = _PJ * _TOT
_SEL[1:_ROWS:2] = _PJ
_ADD[1:_ROWS:2] = _PI * _TOT

_mesh = plsc.VectorSubcoreMesh(core_axis_name="c", subcore_axis_name="s")


@functools.partial(
    pl.kernel,
    mesh=_mesh,
    compiler_params=pltpu.CompilerParams(
        needs_layout_passes=False, use_tc_tiling_on_sc=False),
    out_type=jax.ShapeDtypeStruct((_B,), jnp.float32),
    scratch_types=[
        pltpu.VMEM((2, _GW), jnp.int32),
        pltpu.VMEM((2, _GW, _D), jnp.float32),
        pltpu.VMEM((_BPT * 32,), jnp.int32),
        pltpu.VMEM((_RPE,), jnp.int32),
        pltpu.VMEM((_RPE,), jnp.int32),
        pltpu.VMEM((_TOT,), jnp.float32),
        pltpu.VMEM((_BPT,), jnp.float32),
        pltpu.SemaphoreType.DMA,
        pltpu.SemaphoreType.DMA,
    ],
)
def _ffm_sc(tab_hbm, xo_hbm, lin_hbm, sel_hbm, add_hbm, out_hbm,
            idx_v, rows_v, xo_v, sel_v, add_v, lin_v, out_v,
            gsem0, gsem1):
    wid = lax.axis_index("s") * 2 + lax.axis_index("c")
    gsems = (gsem0, gsem1)

    pltpu.sync_copy(xo_hbm.at[pl.ds(wid * (_BPT * 32), _BPT * 32)], xo_v)
    pltpu.sync_copy(sel_hbm, sel_v)
    pltpu.sync_copy(add_hbm, add_v)
    pltpu.sync_copy(lin_hbm, lin_v)

    lanes = lax.iota(jnp.int32, 16)
    lin_mask = lanes < (_NF - 16)

    def build_idx(g, b):
        gc = jnp.minimum(g, _NG - 1)
        for e in range(_EG):
            b32 = jnp.full((16,), 32, jnp.int32) * (gc * _EG + e)
            for k in range(_RPE // 16):
                sel = sel_v[pl.ds(16 * k, 16)] + b32
                iv = plsc.load_gather(xo_v, [sel]) + add_v[pl.ds(16 * k, 16)]
                idx_v[b, pl.ds(e * _RPE + 16 * k, 16)] = iv

    def fire_gather(b):
        for e in range(_EG):
            for co, cn in _CHUNKS:
                o = e * _RPE + co
                pltpu.async_copy(
                    tab_hbm.at[idx_v.at[b, pl.ds(o, cn)]],
                    rows_v.at[b, pl.ds(o, cn)], gsems[b])

    def wait_gather(b):
        for e in range(_EG):
            for co, cn in _CHUNKS:
                o = e * _RPE + co
                pltpu.make_async_copy(
                    tab_hbm.at[idx_v.at[b, pl.ds(o, cn)]],
                    rows_v.at[b, pl.ds(o, cn)], gsems[b]).wait()

    def compute(g, b):
        for e in range(_EG):
            base = e * _RPE

            def chunk(pc, accs):
                news = list(accs)
                cb = base + 2 * 65 * pc
                for q in range(65):
                    r1 = rows_v[b, cb + 2 * q]
                    r2 = rows_v[b, cb + 2 * q + 1]
                    news[q % 4] = news[q % 4] + r1 * r2
                return tuple(news)

            zv16 = jnp.zeros((16,), jnp.float32)
            a0, a1, a2, a3 = lax.fori_loop(0, _NPAIR // 65, chunk,
                                           (zv16, zv16, zv16, zv16))
            acc = (a0 + a1) + (a2 + a3)
            le = g * _EG + e
            i0 = xo_v[pl.ds(le * 32, 16)]
            i1 = xo_v[pl.ds(le * 32 + 16, 16)]
            l0 = plsc.load_gather(lin_v, [i0])
            l1 = plsc.load_gather(lin_v, [i1])
            zv = acc + l0 + jnp.where(lin_mask, l1, 0.0)
            zc = plsc.cumsum(zv)
            plsc.store_scatter(out_v, [jnp.full((16,), le, jnp.int32)],
                               zc, mask=lanes == 15)

    build_idx(0, 0)
    fire_gather(0)

    def pair_body(gg, carry):
        for bb in range(2):
            g = 2 * gg + bb
            nb = 1 - bb
            build_idx(g + 1, nb)
            fire_gather(nb)
            wait_gather(bb)
            compute(g, bb)
        return carry

    lax.fori_loop(0, _NG // 2, pair_body, 0)
    wait_gather(0)

    for k in range(_BPT // 16):
        z = out_v[pl.ds(k * 16, 16)]
        out_v[pl.ds(k * 16, 16)] = 1.0 / (1.0 + jnp.exp(-z))
    pltpu.sync_copy(out_v, out_hbm.at[pl.ds(wid * _BPT, _BPT)])


def kernel(x, linear_table, cross_tables, bias):
    offsets = jnp.asarray(np.arange(_NF, dtype=np.int32) * _VOC)
    xo = x + offsets[None, :]
    xof = jnp.pad(xo, ((0, 0), (0, 32 - _NF))).reshape(-1)
    lin2 = linear_table[:, 0] + bias[0] / float(_NF)
    tab = cross_tables.reshape(_NF * _TOT, _D)
    sel = jnp.asarray(_SEL)
    add = jnp.asarray(_ADD)
    out = _ffm_sc(tab, xof, lin2, sel, add)
    return out[:, None]
